# baseline (speedup 1.0000x reference)
import functools

import jax
import jax.numpy as jnp
from jax import lax
from jax.experimental import pallas as pl
from jax.experimental.pallas import tpu as pltpu
from jax.experimental.pallas import tpu_sc as plsc

_BLOCK = 512

_NC = 2
_NS = 16
_NW = _NC * _NS
_GCHUNK = 128


def _argmin_kernel(x_ref, cb_ref, x2_ref, c2_ref, idx_ref, cbm2_ref):
    i = pl.program_id(0)

    @pl.when(i == 0)
    def _():
        cbm2_ref[...] = cb_ref[...] * -2.0

    x = x_ref[...]
    x2 = x2_ref[...]
    xcm2 = jax.lax.dot_general(
        x, cbm2_ref[...], (((1,), (1,)), ((), ())),
        preferred_element_type=jnp.float32)
    d2 = x2 + c2_ref[...] + xcm2
    k = d2.shape[1]
    m = jnp.min(d2, axis=-1, keepdims=True)
    s = jnp.sqrt(jnp.maximum(m, 0.0))
    pb = jax.lax.bitcast_convert_type(s * s, jnp.int32)
    hi = m
    for j in (-1, 0, 1, 2):
        vj = jax.lax.bitcast_convert_type(pb + j, jnp.float32)
        ok = jnp.sqrt(vj) == s
        hi = jnp.where(ok, jnp.maximum(hi, vj), hi)
    iota = jax.lax.broadcasted_iota(jnp.int32, d2.shape, 1)
    idx = jnp.min(jnp.where(d2 <= hi, iota, k), axis=-1)
    idx_ref[...] = idx.astype(jnp.int32).reshape(1, 1, idx.shape[0])


def _tc_argmin(x, codebook):
    n, d = x.shape
    k = codebook.shape[0]
    grid = n // _BLOCK
    idx3 = pl.pallas_call(
        _argmin_kernel,
        grid=(grid,),
        in_specs=[
            pl.BlockSpec((_BLOCK, d), lambda i: (i, 0)),
            pl.BlockSpec((k, d), lambda i: (0, 0)),
            pl.BlockSpec((_BLOCK, 1), lambda i: (i, 0)),
            pl.BlockSpec((1, k), lambda i: (0, 0)),
        ],
        out_specs=pl.BlockSpec((1, 1, _BLOCK), lambda i: (i, 0, 0)),
        out_shape=jax.ShapeDtypeStruct((grid, 1, _BLOCK), jnp.int32),
        scratch_shapes=[
            pltpu.VMEM((k, d), jnp.float32),
        ],
    )(x, codebook,
      jnp.sum(x * x, axis=-1, keepdims=True),
      jnp.sum(codebook * codebook, axis=-1)[None, :])
    return idx3.reshape(n)


def _sc_gather(table, idx2d, n, d):
    rows_per_w = n // _NW
    ichunks = rows_per_w // _GCHUNK
    mesh = plsc.VectorSubcoreMesh(
        core_axis_name="c", subcore_axis_name="s",
        num_cores=_NC, num_subcores=_NS)

    @functools.partial(
        pl.kernel, mesh=mesh,
        out_type=jax.ShapeDtypeStruct((n, d), jnp.float32),
        compiler_params=pltpu.CompilerParams(use_tc_tiling_on_sc=False),
        scratch_types=[
            pltpu.VMEM((ichunks, _GCHUNK), jnp.int32),
            pltpu.VMEM((rows_per_w, d), jnp.float32),
            pltpu.SemaphoreType.DMA,
        ],
    )
    def gk(table_hbm, idx_hbm, out_hbm, idx_v, rows_v, sem):
        wid = lax.axis_index("s") * _NC + lax.axis_index("c")
        pltpu.sync_copy(idx_hbm.at[pl.ds(wid * ichunks, ichunks)], idx_v)
        copies = []
        for j in range(ichunks):
            copies.append(pltpu.async_copy(
                table_hbm.at[idx_v.at[j]],
                rows_v.at[pl.ds(j * _GCHUNK, _GCHUNK)],
                sem))
        for c in copies:
            c.wait()
        pltpu.sync_copy(rows_v, out_hbm.at[pl.ds(wid * rows_per_w,
                                                 rows_per_w)])

    return gk(table, idx2d)


def kernel(x, codebook):
    n, d = x.shape
    idx = _tc_argmin(x, codebook)
    q = _sc_gather(codebook, idx.reshape(n // _GCHUNK, _GCHUNK), n, d)
    return q, idx


# speedup vs baseline: 1.3797x; 1.3797x over previous
import functools

import jax
import jax.numpy as jnp
from jax import lax
from jax.experimental import pallas as pl
from jax.experimental.pallas import tpu as pltpu
from jax.experimental.pallas import tpu_sc as plsc

_BLOCK = 512

_NC = 2
_NS = 16
_NW = _NC * _NS
_GCHUNK = 128


def _argmin_kernel(x_ref, cb_ref, x2_ref, c2_ref, idx_ref, cbm2_ref):
    i = pl.program_id(0)

    @pl.when(i == 0)
    def _():
        cbm2_ref[...] = cb_ref[...] * -2.0

    x = x_ref[...]
    b = x.shape[0]
    x2 = x2_ref[...].reshape(1, b)
    c2t = c2_ref[...]
    xcm2t = jax.lax.dot_general(
        cbm2_ref[...], x, (((1,), (1,)), ((), ())),
        preferred_element_type=jnp.float32)
    d2t = (x2 + c2t) + xcm2t
    d2c = jnp.maximum(d2t, 0.0)
    k = d2c.shape[0]
    bits = lax.bitcast_convert_type(d2c, jnp.int32)
    t = bits.reshape(k // 8, 8, b)
    while t.shape[0] > 1:
        h = t.shape[0] // 2
        t = jnp.minimum(t[:h], t[h:])
    mb = jnp.min(t, axis=1)
    mf = lax.bitcast_convert_type(mb, jnp.float32)
    s = jnp.sqrt(mf)
    pb = lax.bitcast_convert_type(s * s, jnp.int32)
    hib = mb
    for j in (-1, 0, 1, 2):
        vj = lax.bitcast_convert_type(pb + j, jnp.float32)
        ok = jnp.sqrt(vj) == s
        hib = jnp.where(ok, jnp.maximum(hib, pb + j), hib)
    t8 = bits.reshape(k // 8, 8, b)
    hit = t8 <= hib[:, None, :]
    cand = jnp.where(
        hit, lax.broadcasted_iota(jnp.int32, t8.shape, 0), k // 8)
    while cand.shape[0] > 1:
        h = cand.shape[0] // 2
        cand = jnp.minimum(cand[:h], cand[h:])
    jmin = cand[0]
    k_cand = jmin * 8 + lax.broadcasted_iota(jnp.int32, jmin.shape, 0)
    idx = jnp.min(k_cand, axis=0)
    idx_ref[...] = idx.reshape(1, 1, b)


def _tc_argmin(x, codebook):
    n, d = x.shape
    k = codebook.shape[0]
    grid = n // _BLOCK
    idx3 = pl.pallas_call(
        _argmin_kernel,
        grid=(grid,),
        in_specs=[
            pl.BlockSpec((_BLOCK, d), lambda i: (i, 0)),
            pl.BlockSpec((k, d), lambda i: (0, 0)),
            pl.BlockSpec((1, 1, _BLOCK), lambda i: (i, 0, 0)),
            pl.BlockSpec((k, 1), lambda i: (0, 0)),
        ],
        out_specs=pl.BlockSpec((1, 1, _BLOCK), lambda i: (i, 0, 0)),
        out_shape=jax.ShapeDtypeStruct((grid, 1, _BLOCK), jnp.int32),
        scratch_shapes=[
            pltpu.VMEM((k, d), jnp.float32),
        ],
    )(x, codebook,
      jnp.sum(x * x, axis=-1, keepdims=True).reshape(grid, 1, _BLOCK),
      jnp.sum(codebook * codebook, axis=-1)[:, None])
    return idx3.reshape(n)


def _sc_gather(table, idx2d, n, d):
    rows_per_w = n // _NW
    ichunks = rows_per_w // _GCHUNK
    mesh = plsc.VectorSubcoreMesh(
        core_axis_name="c", subcore_axis_name="s",
        num_cores=_NC, num_subcores=_NS)

    @functools.partial(
        pl.kernel, mesh=mesh,
        out_type=jax.ShapeDtypeStruct((n, d), jnp.float32),
        compiler_params=pltpu.CompilerParams(use_tc_tiling_on_sc=False),
        scratch_types=[
            pltpu.VMEM((ichunks, _GCHUNK), jnp.int32),
            pltpu.VMEM((rows_per_w, d), jnp.float32),
            pltpu.SemaphoreType.DMA,
        ],
    )
    def gk(table_hbm, idx_hbm, out_hbm, idx_v, rows_v, sem):
        wid = lax.axis_index("s") * _NC + lax.axis_index("c")
        pltpu.sync_copy(idx_hbm.at[pl.ds(wid * ichunks, ichunks)], idx_v)
        copies = []
        for j in range(ichunks):
            copies.append(pltpu.async_copy(
                table_hbm.at[idx_v.at[j]],
                rows_v.at[pl.ds(j * _GCHUNK, _GCHUNK)],
                sem))
        for c in copies:
            c.wait()
        pltpu.sync_copy(rows_v, out_hbm.at[pl.ds(wid * rows_per_w,
                                                 rows_per_w)])

    return gk(table, idx2d)


def kernel(x, codebook):
    n, d = x.shape
    idx = _tc_argmin(x, codebook)
    q = _sc_gather(codebook, idx.reshape(n // _GCHUNK, _GCHUNK), n, d)
    return q, idx


# speedup vs baseline: 1.4136x; 1.0246x over previous
import functools

import jax
import jax.numpy as jnp
from jax import lax
from jax.experimental import pallas as pl
from jax.experimental.pallas import tpu as pltpu
from jax.experimental.pallas import tpu_sc as plsc

_BLOCK = 512

_NC = 2
_NS = 16
_NW = _NC * _NS
_GCHUNK = 128


def _argmin_kernel(x_ref, cb_ref, x2_ref, c2_ref, idx_ref, cbm2_ref):
    i = pl.program_id(0)

    @pl.when(i == 0)
    def _():
        cbm2_ref[...] = cb_ref[...] * -2.0

    x = x_ref[...]
    b = x.shape[0]
    x2 = x2_ref[...].reshape(1, b)
    c2t = c2_ref[...]
    xcm2t = jax.lax.dot_general(
        cbm2_ref[...], x, (((1,), (1,)), ((), ())),
        preferred_element_type=jnp.float32)
    d2t = (x2 + c2t) + xcm2t
    d2c = jnp.maximum(d2t, 0.0)
    k = d2c.shape[0]
    t = d2c.reshape(k // 8, 8, b)
    while t.shape[0] > 1:
        if t.shape[0] % 4 == 0:
            q = t.shape[0] // 4
            t = jnp.minimum(jnp.minimum(t[:q], t[q:2 * q]),
                            jnp.minimum(t[2 * q:3 * q], t[3 * q:]))
        else:
            h = t.shape[0] // 2
            t = jnp.minimum(t[:h], t[h:])
    m = jnp.min(t, axis=1)
    s = jnp.sqrt(m)
    pb = lax.bitcast_convert_type(s * s, jnp.int32)
    hi = m
    for j in (-1, 0, 1, 2):
        vj = lax.bitcast_convert_type(pb + j, jnp.float32)
        ok = jnp.sqrt(vj) == s
        hi = jnp.where(ok, jnp.maximum(hi, vj), hi)
    t8 = d2c.reshape(k // 8, 8, b)
    hit = t8 <= hi[:, None, :]
    cand = jnp.where(
        hit, lax.broadcasted_iota(jnp.int32, t8.shape, 0).astype(jnp.float32),
        jnp.float32(k // 8))
    while cand.shape[0] > 1:
        if cand.shape[0] % 4 == 0:
            q = cand.shape[0] // 4
            cand = jnp.minimum(jnp.minimum(cand[:q], cand[q:2 * q]),
                               jnp.minimum(cand[2 * q:3 * q], cand[3 * q:]))
        else:
            h = cand.shape[0] // 2
            cand = jnp.minimum(cand[:h], cand[h:])
    jmin = cand[0]
    k_cand = (jmin * 8.0 +
              lax.broadcasted_iota(jnp.int32, jmin.shape, 0).astype(jnp.float32))
    idx = jnp.min(k_cand, axis=0)
    idx_ref[...] = idx.astype(jnp.int32).reshape(1, 1, b)


def _tc_argmin(x, codebook):
    n, d = x.shape
    k = codebook.shape[0]
    grid = n // _BLOCK
    idx3 = pl.pallas_call(
        _argmin_kernel,
        grid=(grid,),
        in_specs=[
            pl.BlockSpec((_BLOCK, d), lambda i: (i, 0)),
            pl.BlockSpec((k, d), lambda i: (0, 0)),
            pl.BlockSpec((1, 1, _BLOCK), lambda i: (i, 0, 0)),
            pl.BlockSpec((k, 1), lambda i: (0, 0)),
        ],
        out_specs=pl.BlockSpec((1, 1, _BLOCK), lambda i: (i, 0, 0)),
        out_shape=jax.ShapeDtypeStruct((grid, 1, _BLOCK), jnp.int32),
        scratch_shapes=[
            pltpu.VMEM((k, d), jnp.float32),
        ],
    )(x, codebook,
      jnp.sum(x * x, axis=-1, keepdims=True).reshape(grid, 1, _BLOCK),
      jnp.sum(codebook * codebook, axis=-1)[:, None])
    return idx3.reshape(n)


def _sc_gather(table, idx2d, n, d):
    rows_per_w = n // _NW
    ichunks = rows_per_w // _GCHUNK
    mesh = plsc.VectorSubcoreMesh(
        core_axis_name="c", subcore_axis_name="s",
        num_cores=_NC, num_subcores=_NS)

    @functools.partial(
        pl.kernel, mesh=mesh,
        out_type=jax.ShapeDtypeStruct((n, d), jnp.float32),
        compiler_params=pltpu.CompilerParams(use_tc_tiling_on_sc=False),
        scratch_types=[
            pltpu.VMEM((ichunks, _GCHUNK), jnp.int32),
            pltpu.VMEM((rows_per_w, d), jnp.float32),
            pltpu.SemaphoreType.DMA,
        ],
    )
    def gk(table_hbm, idx_hbm, out_hbm, idx_v, rows_v, sem):
        wid = lax.axis_index("s") * _NC + lax.axis_index("c")
        pltpu.sync_copy(idx_hbm.at[pl.ds(wid * ichunks, ichunks)], idx_v)
        copies = []
        for j in range(ichunks):
            copies.append(pltpu.async_copy(
                table_hbm.at[idx_v.at[j]],
                rows_v.at[pl.ds(j * _GCHUNK, _GCHUNK)],
                sem))
        for c in copies:
            c.wait()
        pltpu.sync_copy(rows_v, out_hbm.at[pl.ds(wid * rows_per_w,
                                                 rows_per_w)])

    return gk(table, idx2d)


def kernel(x, codebook):
    n, d = x.shape
    idx = _tc_argmin(x, codebook)
    q = _sc_gather(codebook, idx.reshape(n // _GCHUNK, _GCHUNK), n, d)
    return q, idx


# speedup vs baseline: 1.4978x; 1.0596x over previous
import functools

import jax
import jax.numpy as jnp
from jax import lax
from jax.experimental import pallas as pl
from jax.experimental.pallas import tpu as pltpu
from jax.experimental.pallas import tpu_sc as plsc

_BLOCK = 1024

_NC = 2
_NS = 16
_NW = _NC * _NS
_GCHUNK = 128


def _argmin_kernel(x_ref, cb_ref, x2_ref, c2_ref, idx_ref, cbm2_ref):
    i = pl.program_id(0)

    @pl.when(i == 0)
    def _():
        cbm2_ref[...] = cb_ref[...] * -2.0

    x = x_ref[...]
    b = x.shape[0]
    x2 = x2_ref[...].reshape(1, b)
    c2t = c2_ref[...]
    xcm2t = jax.lax.dot_general(
        cbm2_ref[...], x, (((1,), (1,)), ((), ())),
        preferred_element_type=jnp.float32)
    d2t = (x2 + c2t) + xcm2t
    d2c = jnp.maximum(d2t, 0.0)
    k = d2c.shape[0]
    t = d2c.reshape(k // 8, 8, b)
    while t.shape[0] > 1:
        if t.shape[0] % 4 == 0:
            q = t.shape[0] // 4
            t = jnp.minimum(jnp.minimum(t[:q], t[q:2 * q]),
                            jnp.minimum(t[2 * q:3 * q], t[3 * q:]))
        else:
            h = t.shape[0] // 2
            t = jnp.minimum(t[:h], t[h:])
    m = jnp.min(t, axis=1)
    s = jnp.sqrt(m)
    pb = lax.bitcast_convert_type(s * s, jnp.int32)
    hi = m
    for j in (-1, 0, 1, 2):
        vj = lax.bitcast_convert_type(pb + j, jnp.float32)
        ok = jnp.sqrt(vj) == s
        hi = jnp.where(ok, jnp.maximum(hi, vj), hi)
    t8 = d2c.reshape(k // 8, 8, b)
    hit = t8 <= hi[:, None, :]
    cand = jnp.where(
        hit, lax.broadcasted_iota(jnp.int32, t8.shape, 0).astype(jnp.float32),
        jnp.float32(k // 8))
    while cand.shape[0] > 1:
        if cand.shape[0] % 4 == 0:
            q = cand.shape[0] // 4
            cand = jnp.minimum(jnp.minimum(cand[:q], cand[q:2 * q]),
                               jnp.minimum(cand[2 * q:3 * q], cand[3 * q:]))
        else:
            h = cand.shape[0] // 2
            cand = jnp.minimum(cand[:h], cand[h:])
    jmin = cand[0]
    k_cand = (jmin * 8.0 +
              lax.broadcasted_iota(jnp.int32, jmin.shape, 0).astype(jnp.float32))
    idx = jnp.min(k_cand, axis=0)
    idx_ref[...] = idx.astype(jnp.int32).reshape(1, 1, b)


def _tc_argmin(x, codebook):
    n, d = x.shape
    k = codebook.shape[0]
    grid = n // _BLOCK
    idx3 = pl.pallas_call(
        _argmin_kernel,
        grid=(grid,),
        in_specs=[
            pl.BlockSpec((_BLOCK, d), lambda i: (i, 0)),
            pl.BlockSpec((k, d), lambda i: (0, 0)),
            pl.BlockSpec((1, 1, _BLOCK), lambda i: (i, 0, 0)),
            pl.BlockSpec((k, 1), lambda i: (0, 0)),
        ],
        out_specs=pl.BlockSpec((1, 1, _BLOCK), lambda i: (i, 0, 0)),
        out_shape=jax.ShapeDtypeStruct((grid, 1, _BLOCK), jnp.int32),
        scratch_shapes=[
            pltpu.VMEM((k, d), jnp.float32),
        ],
    )(x, codebook,
      jnp.sum(x * x, axis=-1, keepdims=True).reshape(grid, 1, _BLOCK),
      jnp.sum(codebook * codebook, axis=-1)[:, None])
    return idx3.reshape(n)


def _sc_gather(table, idx2d, n, d):
    rows_per_w = n // _NW
    ichunks = rows_per_w // _GCHUNK
    mesh = plsc.VectorSubcoreMesh(
        core_axis_name="c", subcore_axis_name="s",
        num_cores=_NC, num_subcores=_NS)

    @functools.partial(
        pl.kernel, mesh=mesh,
        out_type=jax.ShapeDtypeStruct((n, d), jnp.float32),
        compiler_params=pltpu.CompilerParams(use_tc_tiling_on_sc=False),
        scratch_types=[
            pltpu.VMEM((ichunks, _GCHUNK), jnp.int32),
            pltpu.VMEM((rows_per_w, d), jnp.float32),
            pltpu.SemaphoreType.DMA,
        ],
    )
    def gk(table_hbm, idx_hbm, out_hbm, idx_v, rows_v, sem):
        wid = lax.axis_index("s") * _NC + lax.axis_index("c")
        pltpu.sync_copy(idx_hbm.at[pl.ds(wid * ichunks, ichunks)], idx_v)
        copies = []
        for j in range(ichunks):
            copies.append(pltpu.async_copy(
                table_hbm.at[idx_v.at[j]],
                rows_v.at[pl.ds(j * _GCHUNK, _GCHUNK)],
                sem))
        for c in copies:
            c.wait()
        pltpu.sync_copy(rows_v, out_hbm.at[pl.ds(wid * rows_per_w,
                                                 rows_per_w)])

    return gk(table, idx2d)


def kernel(x, codebook):
    n, d = x.shape
    idx = _tc_argmin(x, codebook)
    q = _sc_gather(codebook, idx.reshape(n // _GCHUNK, _GCHUNK), n, d)
    return q, idx


# speedup vs baseline: 1.5277x; 1.0200x over previous
import functools

import jax
import jax.numpy as jnp
from jax import lax
from jax.experimental import pallas as pl
from jax.experimental.pallas import tpu as pltpu
from jax.experimental.pallas import tpu_sc as plsc

_BLOCK = 2048

_NC = 2
_NS = 16
_NW = _NC * _NS
_GCHUNK = 128


def _argmin_kernel(x_ref, cb_ref, x2_ref, c2_ref, idx_ref, cbm2_ref):
    i = pl.program_id(0)

    @pl.when(i == 0)
    def _():
        cbm2_ref[...] = cb_ref[...] * -2.0

    x = x_ref[...]
    b = x.shape[0]
    x2 = x2_ref[...].reshape(1, b)
    c2t = c2_ref[...]
    xcm2t = jax.lax.dot_general(
        cbm2_ref[...], x, (((1,), (1,)), ((), ())),
        preferred_element_type=jnp.float32)
    d2t = (x2 + c2t) + xcm2t
    d2c = jnp.maximum(d2t, 0.0)
    k = d2c.shape[0]
    t = d2c.reshape(k // 8, 8, b)
    while t.shape[0] > 1:
        if t.shape[0] % 4 == 0:
            q = t.shape[0] // 4
            t = jnp.minimum(jnp.minimum(t[:q], t[q:2 * q]),
                            jnp.minimum(t[2 * q:3 * q], t[3 * q:]))
        else:
            h = t.shape[0] // 2
            t = jnp.minimum(t[:h], t[h:])
    m = jnp.min(t, axis=1)
    s = jnp.sqrt(m)
    pb = lax.bitcast_convert_type(s * s, jnp.int32)
    hi = m
    for j in (-1, 0, 1, 2):
        vj = lax.bitcast_convert_type(pb + j, jnp.float32)
        ok = jnp.sqrt(vj) == s
        hi = jnp.where(ok, jnp.maximum(hi, vj), hi)
    t8 = d2c.reshape(k // 8, 8, b)
    hit = t8 <= hi[:, None, :]
    cand = jnp.where(
        hit, lax.broadcasted_iota(jnp.int32, t8.shape, 0).astype(jnp.float32),
        jnp.float32(k // 8))
    while cand.shape[0] > 1:
        if cand.shape[0] % 4 == 0:
            q = cand.shape[0] // 4
            cand = jnp.minimum(jnp.minimum(cand[:q], cand[q:2 * q]),
                               jnp.minimum(cand[2 * q:3 * q], cand[3 * q:]))
        else:
            h = cand.shape[0] // 2
            cand = jnp.minimum(cand[:h], cand[h:])
    jmin = cand[0]
    k_cand = (jmin * 8.0 +
              lax.broadcasted_iota(jnp.int32, jmin.shape, 0).astype(jnp.float32))
    idx = jnp.min(k_cand, axis=0)
    idx_ref[...] = idx.astype(jnp.int32).reshape(1, 1, b)


def _tc_argmin(x, codebook):
    n, d = x.shape
    k = codebook.shape[0]
    grid = n // _BLOCK
    idx3 = pl.pallas_call(
        _argmin_kernel,
        grid=(grid,),
        in_specs=[
            pl.BlockSpec((_BLOCK, d), lambda i: (i, 0)),
            pl.BlockSpec((k, d), lambda i: (0, 0)),
            pl.BlockSpec((1, 1, _BLOCK), lambda i: (i, 0, 0)),
            pl.BlockSpec((k, 1), lambda i: (0, 0)),
        ],
        out_specs=pl.BlockSpec((1, 1, _BLOCK), lambda i: (i, 0, 0)),
        out_shape=jax.ShapeDtypeStruct((grid, 1, _BLOCK), jnp.int32),
        scratch_shapes=[
            pltpu.VMEM((k, d), jnp.float32),
        ],
    )(x, codebook,
      jnp.sum(x * x, axis=-1, keepdims=True).reshape(grid, 1, _BLOCK),
      jnp.sum(codebook * codebook, axis=-1)[:, None])
    return idx3.reshape(n)


def _sc_gather(table, idx2d, n, d):
    rows_per_w = n // _NW
    ichunks = rows_per_w // _GCHUNK
    mesh = plsc.VectorSubcoreMesh(
        core_axis_name="c", subcore_axis_name="s",
        num_cores=_NC, num_subcores=_NS)

    @functools.partial(
        pl.kernel, mesh=mesh,
        out_type=jax.ShapeDtypeStruct((n, d), jnp.float32),
        compiler_params=pltpu.CompilerParams(use_tc_tiling_on_sc=False),
        scratch_types=[
            pltpu.VMEM((ichunks, _GCHUNK), jnp.int32),
            pltpu.VMEM((rows_per_w, d), jnp.float32),
            pltpu.SemaphoreType.DMA,
        ],
    )
    def gk(table_hbm, idx_hbm, out_hbm, idx_v, rows_v, sem):
        wid = lax.axis_index("s") * _NC + lax.axis_index("c")
        pltpu.sync_copy(idx_hbm.at[pl.ds(wid * ichunks, ichunks)], idx_v)
        copies = []
        for j in range(ichunks):
            copies.append(pltpu.async_copy(
                table_hbm.at[idx_v.at[j]],
                rows_v.at[pl.ds(j * _GCHUNK, _GCHUNK)],
                sem))
        for c in copies:
            c.wait()
        pltpu.sync_copy(rows_v, out_hbm.at[pl.ds(wid * rows_per_w,
                                                 rows_per_w)])

    return gk(table, idx2d)


def kernel(x, codebook):
    n, d = x.shape
    idx = _tc_argmin(x, codebook)
    q = _sc_gather(codebook, idx.reshape(n // _GCHUNK, _GCHUNK), n, d)
    return q, idx
